# baseline (speedup 1.0000x reference)
import functools

import jax
import jax.numpy as jnp
from jax import lax
from jax.experimental import pallas as pl
from jax.experimental.pallas import tpu as pltpu
from jax.experimental.pallas import tpu_sc as plsc

_LANE = 128


_NW = 32
_CH_ROWS = 4
_COLS = 4096


def _sc_body(x_hbm, o_hbm, in0, in1, ou0, ou1, is0, is1, os0, os1,
             *, rows_per_worker):
    nch = rows_per_worker // _CH_ROWS
    cid = lax.axis_index("c")
    sid = lax.axis_index("s")
    wid = sid * 2 + cid
    row0 = wid * rows_per_worker

    perm = (jax.lax.iota(jnp.int32, 16) ^ 1).reshape(16, 1)
    parityf = (jax.lax.iota(jnp.int32, 16) & 1).astype(jnp.float32)
    zerov = jnp.zeros((16,), jnp.float32)

    ins = (in0, in1)
    outs = (ou0, ou1)
    isems = (is0, is1)
    osems = (os0, os1)

    def chunk_rows(c):
        return pl.ds(row0 + c * _CH_ROWS, _CH_ROWS)

    pltpu.async_copy(x_hbm.at[chunk_rows(0)], in0, is0)
    pltpu.async_copy(x_hbm.at[chunk_rows(1)], in1, is1)

    def pair_body(p, carry):
        for b in range(2):
            ib, ob, isem, osem = ins[b], outs[b], isems[b], osems[b]
            c = 2 * p + b
            @pl.when(p > 0)
            def _():
                pltpu.make_async_copy(ob, o_hbm.at[chunk_rows(c - 2)],
                                      osem).wait()
            pltpu.make_async_copy(x_hbm.at[chunk_rows(c)], ib, isem).wait()

            for r in range(_CH_ROWS):
                def grp_body(k, _, r=r, ib=ib, ob=ob):
                    base = k * 256
                    for g in range(16):
                        off = base + g * 16
                        xv = ib[r, pl.ds(off, 16)]
                        pv = lax.gather(
                            xv, perm,
                            lax.GatherDimensionNumbers(
                                offset_dims=(),
                                collapsed_slice_dims=(0,),
                                start_index_map=(0,)),
                            (1,),
                            mode=lax.GatherScatterMode.PROMISE_IN_BOUNDS)
                        even = parityf == zerov
                        win = (xv > pv) | (even & (xv == pv))
                        ob[r, pl.ds(off, 16)] = jnp.where(win, xv, zerov)
                    return 0
                lax.fori_loop(0, _COLS // 256, grp_body, 0)

            pltpu.async_copy(ob, o_hbm.at[chunk_rows(c)], osem)

            @pl.when(c + 2 < nch)
            def _():
                pltpu.async_copy(x_hbm.at[chunk_rows(c + 2)], ib, isem)
        return carry

    lax.fori_loop(0, nch // 2, pair_body, 0)

    pltpu.make_async_copy(ou0, o_hbm.at[chunk_rows(nch - 2)], os0).wait()
    pltpu.make_async_copy(ou1, o_hbm.at[chunk_rows(nch - 1)], os1).wait()


def _lwta_sc(x2):
    rows = x2.shape[0]
    rows_per_worker = rows // _NW
    mesh = plsc.VectorSubcoreMesh(core_axis_name="c", subcore_axis_name="s")
    fn = pl.kernel(
        functools.partial(_sc_body, rows_per_worker=rows_per_worker),
        out_type=jax.ShapeDtypeStruct((rows, _COLS), jnp.float32),
        mesh=mesh,
        scratch_types=[
            pltpu.VMEM((_CH_ROWS, _COLS), jnp.float32),
            pltpu.VMEM((_CH_ROWS, _COLS), jnp.float32),
            pltpu.VMEM((_CH_ROWS, _COLS), jnp.float32),
            pltpu.VMEM((_CH_ROWS, _COLS), jnp.float32),
            pltpu.SemaphoreType.DMA,
            pltpu.SemaphoreType.DMA,
            pltpu.SemaphoreType.DMA,
            pltpu.SemaphoreType.DMA,
        ],
    )
    return fn(x2)


_ROWS_PER_BLOCK = 512


def _lwta_body(x_ref, o_ref):
    n = x_ref.shape[1]
    shape = (x_ref.shape[0], _LANE)
    lane = jax.lax.broadcasted_iota(jnp.int32, shape, dimension=1)
    even = (lane & 1) == 0
    odd = ~even
    zero = jnp.zeros(shape, x_ref.dtype)
    for j in range(n // _LANE):
        sl = pl.ds(j * _LANE, _LANE)
        x = x_ref[:, sl]
        left = pltpu.roll(x, _LANE - 1, axis=1)
        right = pltpu.roll(x, 1, axis=1)
        win = (even & (x >= left)) | (odd & (x > right))
        o_ref[:, sl] = jnp.where(win, x, zero)


def _lwta_tc(x2):
    rows = x2.shape[0]
    n_last = x2.shape[1]
    block = _ROWS_PER_BLOCK
    grid = (rows // block,)
    return pl.pallas_call(
        _lwta_body,
        grid=grid,
        in_specs=[pl.BlockSpec((block, n_last), lambda i: (i, 0))],
        out_specs=pl.BlockSpec((block, n_last), lambda i: (i, 0)),
        out_shape=jax.ShapeDtypeStruct((rows, n_last), x2.dtype),
    )(x2)


def kernel(x):
    orig_shape = x.shape
    x2 = x.reshape(-1, orig_shape[-1])
    out = _lwta_sc(x2)
    return out.reshape(orig_shape)
